# baseline (speedup 1.0000x reference)
import functools

import jax
import jax.numpy as jnp
from jax import lax
from jax.experimental import pallas as pl
from jax.experimental.pallas import tpu as pltpu
from jax.experimental.pallas import tpu_sc as plsc

EMBED = 64
MAXCHUNK = 128
NSET = 4
QROWS = 32


def _splits(seq):
    out = []
    off = 0
    while off < seq:
        size = min(MAXCHUNK, seq - off)
        out.append((off, size))
        off += size
    return tuple(out)


@functools.lru_cache(maxsize=None)
def _build(nw, nc, rows_per_w, seq):
    n = nw * rows_per_w * seq
    pieces = _splits(seq)
    nbody = rows_per_w // NSET
    stage_every = QROWS // NSET
    wbytes = seq * EMBED * 4
    mesh = plsc.VectorSubcoreMesh(core_axis_name="c", subcore_axis_name="s")

    @functools.partial(
        pl.kernel,
        mesh=mesh,
        compiler_params=pltpu.CompilerParams(use_tc_tiling_on_sc=False),
        out_type=jax.ShapeDtypeStruct((2 * n, EMBED), jnp.float32),
        scratch_types=[
            pltpu.VMEM((QROWS, seq), jnp.int32),
            pltpu.VMEM((QROWS, seq), jnp.int32),
            pltpu.VMEM((NSET, seq, EMBED), jnp.float32),
            pltpu.VMEM((NSET, seq, EMBED), jnp.float32),
            pltpu.SemaphoreType.DMA,
            pltpu.SemaphoreType.DMA,
            pltpu.SemaphoreType.DMA,
            pltpu.SemaphoreType.DMA,
            pltpu.SemaphoreType.DMA,
        ],
    )
    def emb(txt, syl, w_char, w_syl, out, idxc_q, idxs_q, bufc, bufs,
            gsem, w0, w1, w2, w3):
        wsems = (w0, w1, w2, w3)
        wid = lax.axis_index("s") * nc + lax.axis_index("c")
        row0 = wid * rows_per_w

        def drain(s):
            pltpu.make_async_copy(
                bufc.at[s], out.at[pl.ds(0, seq), pl.ds(0, EMBED)],
                wsems[s]).wait()
            pltpu.make_async_copy(
                bufs.at[s], out.at[pl.ds(0, seq), pl.ds(EMBED, EMBED)],
                wsems[s]).wait()

        def body(j, carry):
            @pl.when(j % stage_every == 0)
            def _stage():
                q0 = row0 + j * NSET
                pltpu.sync_copy(txt.at[pl.ds(q0, QROWS)], idxc_q)
                pltpu.sync_copy(syl.at[pl.ds(q0, QROWS)], idxs_q)

            wcps = []
            for s in range(NSET):
                row = (row0 + j * NSET + s) * seq
                wcps.append(pltpu.async_copy(
                    bufc.at[s], out.at[pl.ds(2 * row, seq)],
                    wsems[s]))
                wcps.append(pltpu.async_copy(
                    bufs.at[s], out.at[pl.ds(2 * row + seq, seq)],
                    wsems[s]))
            for w in wcps:
                w.wait()
            return carry

        lax.fori_loop(0, nbody, body, 0)

    return emb


def kernel(txt_input, syl_input, W_char, W_syl):
    b, s = txt_input.shape
    info = plsc.get_sparse_core_info()
    nc, ns = info.num_cores, info.num_subcores
    nw = nc * ns
    emb = _build(nw, nc, b // nw, s)
    out = emb(txt_input.astype(jnp.int32), syl_input.astype(jnp.int32),
              W_char, W_syl)
    return out.reshape(b, s, 2 * EMBED)
